# baseline (speedup 1.0000x reference)
import functools

import jax
import jax.numpy as jnp
from jax import lax
from jax.experimental import pallas as pl
from jax.experimental.pallas import tpu as pltpu
from jax.experimental.pallas import tpu_sc as plsc

B = 16384
DIM = 64
NCTX = 5
NC = 2
NS = 16
NW = NC * NS
BPW = B // NW
CH = 128
NCHUNK = BPW // CH
LANES = 16
NG = CH // LANES
VHALF = 500000


def _make_kernel():
    mesh = plsc.VectorSubcoreMesh(core_axis_name="c", subcore_axis_name="s")

    @functools.partial(
        pl.kernel,
        out_type=jax.ShapeDtypeStruct((B * NCTX,), jnp.float32),
        mesh=mesh,
        compiler_params=pltpu.CompilerParams(needs_layout_passes=False),
        scratch_types=[
            pltpu.VMEM((1, CH), jnp.int32),
            pltpu.VMEM((NCTX, CH), jnp.int32),
            pltpu.VMEM((CH,), jnp.int32),
            pltpu.VMEM((CH * NCTX,), jnp.int32),
            pltpu.VMEM((CH, 2 * DIM), jnp.float32),
            pltpu.VMEM((CH * NCTX, 2 * DIM), jnp.float32),
            pltpu.VMEM((CH * NCTX,), jnp.float32),
            pltpu.SemaphoreType.DMA,
        ],
    )
    def body(tgt_hbm, ctx_hbm, ttab_hbm, ctab_hbm, out_hbm,
             tgt_idx, ctx_idx, tgt_par, ctx_par, tgt_rows, ctx_rows,
             out_v, sem):
        wid = lax.axis_index("s") * NC + lax.axis_index("c")
        lane = lax.iota(jnp.int32, LANES)

        @pl.loop(0, NCHUNK)
        def _chunk(ch):
            rowb = wid * NCHUNK + ch
            pltpu.sync_copy(tgt_hbm.at[pl.ds(rowb * CH, CH)], tgt_idx.at[0])
            for j in range(NCTX):
                pltpu.sync_copy(
                    ctx_hbm.at[pl.ds(rowb * NCTX * CH + j * CH, CH)],
                    ctx_idx.at[j])
            for v in range(CH // LANES):
                raw = tgt_idx[0, pl.ds(v * LANES, LANES)]
                tgt_par[pl.ds(v * LANES, LANES)] = (raw & 1) * DIM
                tgt_idx[0, pl.ds(v * LANES, LANES)] = raw >> 1
            for j in range(NCTX):
                for v in range(CH // LANES):
                    raw = ctx_idx[j, pl.ds(v * LANES, LANES)]
                    ctx_par[pl.ds(j * CH + v * LANES, LANES)] = (raw & 1) * DIM
                    ctx_idx[j, pl.ds(v * LANES, LANES)] = raw >> 1

            cps = [pltpu.async_copy(ttab_hbm.at[tgt_idx.at[0]], tgt_rows, sem)]
            for j in range(NCTX):
                cps.append(pltpu.async_copy(
                    ctab_hbm.at[ctx_idx.at[j]],
                    ctx_rows.at[pl.ds(j * CH, CH)], sem))
            for cp in cps:
                cp.wait()

            @pl.loop(0, NG)
            def _grp(g):
                rb = g * LANES
                trow = rb + lane
                tcol0 = plsc.load_gather(tgt_par, [trow])
                crows, ccol0, accs = [], [], []
                for c in range(NCTX):
                    p = trow * NCTX + c
                    crows.append(p)
                    ccol0.append(plsc.load_gather(ctx_par, [p]))
                    accs.append(jnp.zeros((LANES,), jnp.float32))
                for d in range(DIM):
                    tv = plsc.load_gather(tgt_rows, [trow, tcol0 + d])
                    for c in range(NCTX):
                        cv = plsc.load_gather(ctx_rows,
                                              [crows[c], ccol0[c] + d])
                        accs[c] = accs[c] + cv * tv
                for c in range(NCTX):
                    plsc.store_scatter(out_v, [crows[c]], accs[c])

            pltpu.sync_copy(out_v,
                            out_hbm.at[pl.ds(rowb * CH * NCTX, CH * NCTX)])

    return body


_sc_kernel = _make_kernel()


def kernel(target, context, target_table, context_table):
    tgt1 = target.reshape(B).astype(jnp.int32)
    ctx1 = context.reshape(B * NCTX).astype(jnp.int32)
    ttab = target_table.reshape(VHALF, 2 * DIM)
    ctab = context_table.reshape(VHALF, 2 * DIM)
    flat = _sc_kernel(tgt1, ctx1, ttab, ctab)
    return flat.reshape(B, NCTX)


# speedup vs baseline: 1.0830x; 1.0830x over previous
import functools

import jax
import jax.numpy as jnp
from jax import lax
from jax.experimental import pallas as pl
from jax.experimental.pallas import tpu as pltpu
from jax.experimental.pallas import tpu_sc as plsc

B = 16384
DIM = 64
NCTX = 5
NC = 2
NS = 16
NW = NC * NS
BPW = B // NW
CH = 256
NCHUNK = BPW // CH
LANES = 16
NG = CH // LANES
IB = CH // 128


def _vsum(vs):
    vs = list(vs)
    while len(vs) > 1:
        vs = [a + b for a, b in zip(vs[::2], vs[1::2])] + (
            [vs[-1]] if len(vs) % 2 else [])
    return vs[0]


def _make_kernel():
    mesh = plsc.VectorSubcoreMesh(core_axis_name="c", subcore_axis_name="s")

    @functools.partial(
        pl.kernel,
        out_type=jax.ShapeDtypeStruct((B * NCTX,), jnp.float32),
        mesh=mesh,
        compiler_params=pltpu.CompilerParams(needs_layout_passes=False,
                                             use_tc_tiling_on_sc=False),
        scratch_types=[
            pltpu.VMEM((IB, 128), jnp.int32),
            pltpu.VMEM((NCTX * IB, 128), jnp.int32),
            pltpu.VMEM((CH, DIM), jnp.float32),
            pltpu.VMEM((CH * NCTX, DIM), jnp.float32),
            pltpu.VMEM((NCTX * LANES * LANES,), jnp.float32),
            pltpu.VMEM((CH * NCTX,), jnp.float32),
            pltpu.SemaphoreType.DMA,
            pltpu.SemaphoreType.DMA,
        ],
    )
    def body(tgt_hbm, ctx_hbm, ttab_hbm, ctab_hbm, out_hbm,
             tgt_idx, ctx_idx, tgt_rows, ctx_rows, part, out_v, sem, sem2):
        wid = lax.axis_index("s") * NC + lax.axis_index("c")
        lane = lax.iota(jnp.int32, LANES)

        @pl.loop(0, NCHUNK)
        def _chunk(ch):
            base = (wid * NCHUNK + ch) * CH
            icps = [pltpu.async_copy(
                tgt_hbm.at[pl.ds(base + j * 128, 128)], tgt_idx.at[j], sem2)
                for j in range(IB)]
            icps += [pltpu.async_copy(
                ctx_hbm.at[pl.ds(base * NCTX + j * 128, 128)], ctx_idx.at[j],
                sem2) for j in range(NCTX * IB)]
            for cp in icps:
                cp.wait()
            cps = [pltpu.async_copy(
                ttab_hbm.at[tgt_idx.at[j]],
                tgt_rows.at[pl.ds(j * 128, 128)], sem) for j in range(IB)]
            cps += [pltpu.async_copy(
                ctab_hbm.at[ctx_idx.at[j]],
                ctx_rows.at[pl.ds(j * 128, 128)], sem)
                for j in range(NCTX * IB)]
            for cp in cps:
                cp.wait()

            @pl.loop(0, NG)
            def _grp(g):
                rb = g * LANES
                for r in range(LANES):
                    rr = rb + r
                    t = [tgt_rows[rr, pl.ds(k * LANES, LANES)]
                         for k in range(4)]
                    widx = lane * LANES + r
                    for c in range(NCTX):
                        p = rr * NCTX + c
                        acc = _vsum([
                            ctx_rows[p, pl.ds(k * LANES, LANES)] * t[k]
                            for k in range(4)])
                        plsc.store_scatter(part, [widx + c * LANES * LANES],
                                           acc)
                for c in range(NCTX):
                    s = _vsum([part[pl.ds(c * LANES * LANES + l * LANES,
                                          LANES)]
                               for l in range(LANES)])
                    oidx = lane * NCTX + (rb * NCTX + c)
                    plsc.store_scatter(out_v, [oidx], s)

            pltpu.sync_copy(out_v,
                            out_hbm.at[pl.ds(base * NCTX, CH * NCTX)])

    return body


_sc_kernel = _make_kernel()


def kernel(target, context, target_table, context_table):
    tgt1 = target.reshape(B).astype(jnp.int32)
    ctx1 = context.reshape(B * NCTX).astype(jnp.int32)
    flat = _sc_kernel(tgt1, ctx1, target_table, context_table)
    return flat.reshape(B, NCTX)
